# Initial kernel scaffold; baseline (speedup 1.0000x reference)
import jax
import jax.numpy as jnp
from jax.experimental import pallas as pl


def kernel(x, edge_index, Wl1, Wr1, b1, Wl2, Wr2, b2, Wl3, Wr3, b3):
    raise NotImplementedError("write your pallas kernel here")


# speedup vs baseline: 2.7800x; 2.7800x over previous
import jax
import jax.numpy as jnp
from jax import lax
from jax.experimental import pallas as pl
from jax.experimental.pallas import tpu as pltpu
from jax.experimental.pallas import tpu_sc as plsc

D = 128
B = 128
CI = 8
NC = 2
NS = 16
NW = NC * NS


def _sc_mesh():
    return plsc.VectorSubcoreMesh(core_axis_name="c", subcore_axis_name="s",
                                  num_cores=NC, num_subcores=NS)


def _make_sc_agg(acc_rows, k_steps):
    rpt = acc_rows // NS
    chunks = rpt // B
    n_ci = k_steps // CI

    out_types = [jax.ShapeDtypeStruct((NC, acc_rows, D), jnp.float32)]
    scratch_types = [
        pltpu.VMEM((CI, B), jnp.int32),
        pltpu.VMEM((CI, B), jnp.int32),
        pltpu.VMEM((B, D), jnp.float32),
        pltpu.VMEM_SHARED((acc_rows, D), jnp.float32),
        pltpu.SemaphoreType.DMA,
    ]

    def body(y_hbm, src_hbm, dst_hbm, out_s, src_v, dst_v, rows_v, acc_sh,
             sem):
        cid = lax.axis_index("c")
        sid = lax.axis_index("s")
        gwid = cid * NS + sid
        base = sid * rpt

        zv = jnp.zeros((16,), jnp.float32)

        def fill_row(r, carry):
            for c in range(D // 16):
                rows_v[r, pl.ds(c * 16, 16)] = zv
            return carry

        lax.fori_loop(0, B, fill_row, 0)
        for t in range(chunks):
            pltpu.sync_copy(rows_v, acc_sh.at[pl.ds(base + t * B, B)])
        plsc.subcore_barrier()

        def outer(ci, carry):
            off = gwid * k_steps + ci * CI
            pltpu.sync_copy(src_hbm.at[pl.ds(off, CI)], src_v)
            pltpu.sync_copy(dst_hbm.at[pl.ds(off, CI)], dst_v)
            for j in range(CI):
                pltpu.async_copy(y_hbm.at[src_v.at[j]], rows_v, sem).wait()
                pltpu.sync_copy(rows_v, acc_sh.at[dst_v.at[j]], add=True)
            return carry

        lax.fori_loop(0, n_ci, outer, 0)
        plsc.subcore_barrier()

        for t in range(chunks):
            pltpu.sync_copy(acc_sh.at[pl.ds(base + t * B, B)], rows_v)
            pltpu.sync_copy(rows_v, out_s.at[cid, pl.ds(base + t * B, B)])

    return pl.kernel(body, mesh=_sc_mesh(), out_type=out_types,
                     scratch_types=scratch_types)


def _make_sc_deg(acc_rows, k_steps):
    rpt = acc_rows // NS
    chunks = rpt // B
    n_ci = k_steps // CI

    out_types = [jax.ShapeDtypeStruct((NC, acc_rows, D), jnp.float32)]
    scratch_types = [
        pltpu.VMEM((CI, B), jnp.int32),
        pltpu.VMEM((B, D), jnp.float32),
        pltpu.VMEM_SHARED((acc_rows, D), jnp.float32),
    ]

    def body(dst_hbm, out_d, dst_v, buf_v, deg_sh):
        cid = lax.axis_index("c")
        sid = lax.axis_index("s")
        gwid = cid * NS + sid
        base = sid * rpt

        def fill(val):
            vv = jnp.full((16,), val, dtype=jnp.float32)

            def fill_row(r, carry):
                for c in range(D // 16):
                    buf_v[r, pl.ds(c * 16, 16)] = vv
                return carry

            lax.fori_loop(0, B, fill_row, 0)

        fill(0.0)
        for t in range(chunks):
            pltpu.sync_copy(buf_v, deg_sh.at[pl.ds(base + t * B, B)])
        fill(1.0)
        plsc.subcore_barrier()

        def outer(ci, carry):
            off = gwid * k_steps + ci * CI
            pltpu.sync_copy(dst_hbm.at[pl.ds(off, CI)], dst_v)
            for j in range(CI):
                pltpu.sync_copy(buf_v, deg_sh.at[dst_v.at[j]], add=True)
            return carry

        lax.fori_loop(0, n_ci, outer, 0)
        plsc.subcore_barrier()

        for t in range(chunks):
            pltpu.sync_copy(deg_sh.at[pl.ds(base + t * B, B)], buf_v)
            pltpu.sync_copy(buf_v, out_d.at[cid, pl.ds(base + t * B, B)])

    return pl.kernel(body, mesh=_sc_mesh(), out_type=out_types,
                     scratch_types=scratch_types)


def _pre_body(x_ref, wl_ref, wr_ref, b_ref, y_ref, z_ref):
    xb = x_ref[...]
    y_ref[...] = jnp.dot(xb, wl_ref[...], preferred_element_type=jnp.float32)
    z_ref[...] = (jnp.dot(xb, wr_ref[...], preferred_element_type=jnp.float32)
                  + b_ref[...])


def _mid_body(p0_ref, p1_ref, d0_ref, d1_ref, z_ref, wl_ref, wr_ref, b_ref,
              y_ref, zo_ref):
    deg = d0_ref[...][:, 0:1] + d1_ref[...][:, 0:1]
    inv = 1.0 / jnp.maximum(deg, 1.0)
    h = jnp.maximum((p0_ref[...] + p1_ref[...]) * inv + z_ref[...], 0.0)
    y_ref[...] = jnp.dot(h, wl_ref[...], preferred_element_type=jnp.float32)
    zo_ref[...] = (jnp.dot(h, wr_ref[...], preferred_element_type=jnp.float32)
                   + b_ref[...])


def _final_body(p0_ref, p1_ref, d0_ref, d1_ref, z_ref, o_ref):
    deg = d0_ref[...][:, 0:1] + d1_ref[...][:, 0:1]
    inv = 1.0 / jnp.maximum(deg, 1.0)
    o_ref[...] = (p0_ref[...] + p1_ref[...]) * inv + z_ref[...]


def _row_spec(r):
    return pl.BlockSpec((r, D), lambda i: (i, 0))


def _deg_spec(r):
    return pl.BlockSpec((r, D), lambda i: (i, 0))


def _w_spec():
    return pl.BlockSpec((D, D), lambda i: (0, 0))


def _b_spec():
    return pl.BlockSpec((1, D), lambda i: (0, 0))


def _tc_pre(x, wl, wr, b, blk):
    n = x.shape[0]
    return pl.pallas_call(
        _pre_body,
        grid=(n // blk,),
        in_specs=[_row_spec(blk), _w_spec(), _w_spec(), _b_spec()],
        out_specs=[_row_spec(blk), _row_spec(blk)],
        out_shape=[jax.ShapeDtypeStruct((n, D), jnp.float32)] * 2,
    )(x, wl, wr, b.reshape(1, D))


def _tc_mid(p0, p1, d0, d1, z, wl, wr, b, blk):
    n = z.shape[0]
    return pl.pallas_call(
        _mid_body,
        grid=(n // blk,),
        in_specs=[_row_spec(blk), _row_spec(blk), _deg_spec(blk),
                  _deg_spec(blk), _row_spec(blk), _w_spec(), _w_spec(),
                  _b_spec()],
        out_specs=[_row_spec(blk), _row_spec(blk)],
        out_shape=[jax.ShapeDtypeStruct((n, D), jnp.float32)] * 2,
    )(p0, p1, d0, d1, z, wl, wr, b.reshape(1, D))


def _tc_final(p0, p1, d0, d1, z, blk):
    n = z.shape[0]
    return pl.pallas_call(
        _final_body,
        grid=(n // blk,),
        in_specs=[_row_spec(blk), _row_spec(blk), _deg_spec(blk),
                  _deg_spec(blk), _row_spec(blk)],
        out_specs=_row_spec(blk),
        out_shape=jax.ShapeDtypeStruct((n, D), jnp.float32),
    )(p0, p1, d0, d1, z)


def kernel(x, edge_index, Wl1, Wr1, b1, Wl2, Wr2, b2, Wl3, Wr3, b3):
    n = x.shape[0]
    e = edge_index.shape[1]

    k_steps = -(-e // (NW * B))
    k_steps = -(-k_steps // CI) * CI
    e_pad = NW * B * k_steps
    acc_rows = -(-(n + 1) // (NS * B)) * (NS * B)
    blk = 1000 if n % 1000 == 0 else 8 * (n // 8)

    src = edge_index[0]
    dst = edge_index[1]
    pad = e_pad - e
    if pad:
        src = jnp.concatenate([src, jnp.zeros((pad,), jnp.int32)])
        dst = jnp.concatenate([dst, jnp.full((pad,), n, dtype=jnp.int32)])
    src2 = src.reshape(e_pad // B, B)
    dst2 = dst.reshape(e_pad // B, B)

    sc_agg = _make_sc_agg(acc_rows, k_steps)
    sc_deg = _make_sc_deg(acc_rows, k_steps)

    (dp,) = sc_deg(dst2)
    d0 = dp[0, :n]
    d1 = dp[1, :n]

    y1, z1 = _tc_pre(x, Wl1, Wr1, b1, blk)
    (s1,) = sc_agg(y1, src2, dst2)

    y2, z2 = _tc_mid(s1[0, :n], s1[1, :n], d0, d1, z1, Wl2, Wr2, b2, blk)
    (s2,) = sc_agg(y2, src2, dst2)

    y3, z3 = _tc_mid(s2[0, :n], s2[1, :n], d0, d1, z2, Wl3, Wr3, b3, blk)
    (s3,) = sc_agg(y3, src2, dst2)

    return _tc_final(s3[0, :n], s3[1, :n], d0, d1, z3, blk)


# speedup vs baseline: 2.9119x; 1.0474x over previous
import jax
import jax.numpy as jnp
from jax import lax
from jax.experimental import pallas as pl
from jax.experimental.pallas import tpu as pltpu
from jax.experimental.pallas import tpu_sc as plsc

D = 128
B = 128
BE = 64
CI = 8
CI2 = 16
NC = 2
NS = 16
NW = NC * NS


def _sc_mesh():
    return plsc.VectorSubcoreMesh(core_axis_name="c", subcore_axis_name="s",
                                  num_cores=NC, num_subcores=NS)


def _make_sc_agg(acc_rows, k64):
    rpt = acc_rows // NS
    chunks = rpt // BE
    n_ci = k64 // CI2

    out_types = [jax.ShapeDtypeStruct((NC, acc_rows, D), jnp.float32)]
    scratch_types = [
        pltpu.VMEM((CI2, BE), jnp.int32),
        pltpu.VMEM((CI2, BE), jnp.int32),
        pltpu.VMEM((BE, D), jnp.float32),
        pltpu.VMEM((BE, D), jnp.float32),
        pltpu.VMEM_SHARED((acc_rows, D), jnp.float32),
        pltpu.SemaphoreType.DMA,
        pltpu.SemaphoreType.DMA,
        pltpu.SemaphoreType.DMA,
    ]

    def body(y_hbm, src_hbm, dst_hbm, out_s, src_v, dst_v, rows0, rows1,
             acc_sh, gsem, ssem0, ssem1):
        cid = lax.axis_index("c")
        sid = lax.axis_index("s")
        gwid = cid * NS + sid
        base = sid * rpt
        bufs = (rows0, rows1)
        ssems = (ssem0, ssem1)

        zv = jnp.zeros((16,), jnp.float32)

        def fill_row(r, carry):
            for c in range(D // 16):
                rows0[r, pl.ds(c * 16, 16)] = zv
            return carry

        lax.fori_loop(0, BE, fill_row, 0)
        for t in range(chunks):
            pltpu.sync_copy(rows0, acc_sh.at[pl.ds(base + t * BE, BE)])
        plsc.subcore_barrier()

        def outer(ci, carry):
            off = gwid * k64 + ci * CI2
            pltpu.sync_copy(src_hbm.at[pl.ds(off, CI2)], src_v)
            pltpu.sync_copy(dst_hbm.at[pl.ds(off, CI2)], dst_v)
            pltpu.async_copy(y_hbm.at[src_v.at[0]], bufs[0], gsem).wait()
            pending = [None, None]
            for j in range(CI2):
                p = j % 2
                pending[p] = pltpu.async_copy(
                    bufs[p], acc_sh.at[dst_v.at[j]], ssems[p], add=True)
                if pending[1 - p] is not None:
                    pending[1 - p].wait()
                if j < CI2 - 1:
                    pltpu.async_copy(
                        y_hbm.at[src_v.at[j + 1]], bufs[1 - p], gsem).wait()
            pending[(CI2 - 1) % 2].wait()
            return carry

        lax.fori_loop(0, n_ci, outer, 0)
        plsc.subcore_barrier()

        for t in range(chunks):
            pltpu.sync_copy(acc_sh.at[pl.ds(base + t * BE, BE)], rows0)
            pltpu.sync_copy(rows0, out_s.at[cid, pl.ds(base + t * BE, BE)])

    return pl.kernel(body, mesh=_sc_mesh(), out_type=out_types,
                     scratch_types=scratch_types)


def _make_sc_deg(acc_rows, k_steps):
    rpt = acc_rows // NS
    chunks = rpt // B
    n_ci = k_steps // CI

    out_types = [jax.ShapeDtypeStruct((NC, acc_rows, D), jnp.float32)]
    scratch_types = [
        pltpu.VMEM((CI, B), jnp.int32),
        pltpu.VMEM((B, D), jnp.float32),
        pltpu.VMEM_SHARED((acc_rows, D), jnp.float32),
    ]

    def body(dst_hbm, out_d, dst_v, buf_v, deg_sh):
        cid = lax.axis_index("c")
        sid = lax.axis_index("s")
        gwid = cid * NS + sid
        base = sid * rpt

        def fill(val):
            vv = jnp.full((16,), val, dtype=jnp.float32)

            def fill_row(r, carry):
                for c in range(D // 16):
                    buf_v[r, pl.ds(c * 16, 16)] = vv
                return carry

            lax.fori_loop(0, B, fill_row, 0)

        fill(0.0)
        for t in range(chunks):
            pltpu.sync_copy(buf_v, deg_sh.at[pl.ds(base + t * B, B)])
        fill(1.0)
        plsc.subcore_barrier()

        def outer(ci, carry):
            off = gwid * k_steps + ci * CI
            pltpu.sync_copy(dst_hbm.at[pl.ds(off, CI)], dst_v)
            for j in range(CI):
                pltpu.sync_copy(buf_v, deg_sh.at[dst_v.at[j]], add=True)
            return carry

        lax.fori_loop(0, n_ci, outer, 0)
        plsc.subcore_barrier()

        for t in range(chunks):
            pltpu.sync_copy(deg_sh.at[pl.ds(base + t * B, B)], buf_v)
            pltpu.sync_copy(buf_v, out_d.at[cid, pl.ds(base + t * B, B)])

    return pl.kernel(body, mesh=_sc_mesh(), out_type=out_types,
                     scratch_types=scratch_types)


def _pre_body(x_ref, wl_ref, wr_ref, b_ref, y_ref, z_ref):
    xb = x_ref[...]
    y_ref[...] = jnp.dot(xb, wl_ref[...], preferred_element_type=jnp.float32)
    z_ref[...] = (jnp.dot(xb, wr_ref[...], preferred_element_type=jnp.float32)
                  + b_ref[...])


def _mid_body(p0_ref, p1_ref, d0_ref, d1_ref, z_ref, wl_ref, wr_ref, b_ref,
              y_ref, zo_ref):
    deg = d0_ref[...][:, 0:1] + d1_ref[...][:, 0:1]
    inv = 1.0 / jnp.maximum(deg, 1.0)
    h = jnp.maximum((p0_ref[...] + p1_ref[...]) * inv + z_ref[...], 0.0)
    y_ref[...] = jnp.dot(h, wl_ref[...], preferred_element_type=jnp.float32)
    zo_ref[...] = (jnp.dot(h, wr_ref[...], preferred_element_type=jnp.float32)
                   + b_ref[...])


def _final_body(p0_ref, p1_ref, d0_ref, d1_ref, z_ref, o_ref):
    deg = d0_ref[...][:, 0:1] + d1_ref[...][:, 0:1]
    inv = 1.0 / jnp.maximum(deg, 1.0)
    o_ref[...] = (p0_ref[...] + p1_ref[...]) * inv + z_ref[...]


def _row_spec(r):
    return pl.BlockSpec((r, D), lambda i: (i, 0))


def _deg_spec(r):
    return pl.BlockSpec((r, D), lambda i: (i, 0))


def _w_spec():
    return pl.BlockSpec((D, D), lambda i: (0, 0))


def _b_spec():
    return pl.BlockSpec((1, D), lambda i: (0, 0))


def _tc_pre(x, wl, wr, b, blk):
    n = x.shape[0]
    return pl.pallas_call(
        _pre_body,
        grid=(n // blk,),
        in_specs=[_row_spec(blk), _w_spec(), _w_spec(), _b_spec()],
        out_specs=[_row_spec(blk), _row_spec(blk)],
        out_shape=[jax.ShapeDtypeStruct((n, D), jnp.float32)] * 2,
    )(x, wl, wr, b.reshape(1, D))


def _tc_mid(p0, p1, d0, d1, z, wl, wr, b, blk):
    n = z.shape[0]
    return pl.pallas_call(
        _mid_body,
        grid=(n // blk,),
        in_specs=[_row_spec(blk), _row_spec(blk), _deg_spec(blk),
                  _deg_spec(blk), _row_spec(blk), _w_spec(), _w_spec(),
                  _b_spec()],
        out_specs=[_row_spec(blk), _row_spec(blk)],
        out_shape=[jax.ShapeDtypeStruct((n, D), jnp.float32)] * 2,
    )(p0, p1, d0, d1, z, wl, wr, b.reshape(1, D))


def _tc_final(p0, p1, d0, d1, z, blk):
    n = z.shape[0]
    return pl.pallas_call(
        _final_body,
        grid=(n // blk,),
        in_specs=[_row_spec(blk), _row_spec(blk), _deg_spec(blk),
                  _deg_spec(blk), _row_spec(blk)],
        out_specs=_row_spec(blk),
        out_shape=jax.ShapeDtypeStruct((n, D), jnp.float32),
    )(p0, p1, d0, d1, z)


def kernel(x, edge_index, Wl1, Wr1, b1, Wl2, Wr2, b2, Wl3, Wr3, b3):
    n = x.shape[0]
    e = edge_index.shape[1]

    k_steps = -(-e // (NW * B))
    k_steps = -(-k_steps // CI) * CI
    e_pad = NW * B * k_steps
    k64 = 2 * k_steps
    acc_rows = -(-(n + 1) // (NS * B)) * (NS * B)
    blk = 1000 if n % 1000 == 0 else 8 * (n // 8)

    src = edge_index[0]
    dst = edge_index[1]
    pad = e_pad - e
    if pad:
        src = jnp.concatenate([src, jnp.zeros((pad,), jnp.int32)])
        dst = jnp.concatenate([dst, jnp.full((pad,), n, dtype=jnp.int32)])
    src64 = src.reshape(e_pad // BE, BE)
    dst64 = dst.reshape(e_pad // BE, BE)
    dst2 = dst.reshape(e_pad // B, B)

    sc_agg = _make_sc_agg(acc_rows, k64)
    sc_deg = _make_sc_deg(acc_rows, k_steps)

    (dp,) = sc_deg(dst2)
    d0 = dp[0, :n]
    d1 = dp[1, :n]

    y1, z1 = _tc_pre(x, Wl1, Wr1, b1, blk)
    (s1,) = sc_agg(y1, src64, dst64)

    y2, z2 = _tc_mid(s1[0, :n], s1[1, :n], d0, d1, z1, Wl2, Wr2, b2, blk)
    (s2,) = sc_agg(y2, src64, dst64)

    y3, z3 = _tc_mid(s2[0, :n], s2[1, :n], d0, d1, z2, Wl3, Wr3, b3, blk)
    (s3,) = sc_agg(y3, src64, dst64)

    return _tc_final(s3[0, :n], s3[1, :n], d0, d1, z3, blk)


# speedup vs baseline: 3.2839x; 1.1277x over previous
import jax
import jax.numpy as jnp
from jax import lax
from jax.experimental import pallas as pl
from jax.experimental.pallas import tpu as pltpu
from jax.experimental.pallas import tpu_sc as plsc

D = 128
B = 128
BE = 32
CI = 8
CI2 = 32
NBUF = 4
NC = 2
NS = 16
NW = NC * NS


def _sc_mesh():
    return plsc.VectorSubcoreMesh(core_axis_name="c", subcore_axis_name="s",
                                  num_cores=NC, num_subcores=NS)


def _make_sc_agg(acc_rows, k64):
    rpt = acc_rows // NS
    chunks = rpt // B
    n_ci = k64 // CI2

    out_types = [jax.ShapeDtypeStruct((NC, acc_rows, D), jnp.float32)]
    scratch_types = [
        pltpu.VMEM((CI2, BE), jnp.int32),
        pltpu.VMEM((CI2, BE), jnp.int32),
        pltpu.VMEM((NBUF * BE, D), jnp.float32),
        pltpu.VMEM_SHARED((acc_rows, D), jnp.float32),
    ] + [pltpu.SemaphoreType.DMA] * (2 * NBUF)

    def body(y_hbm, src_hbm, dst_hbm, out_s, src_v, dst_v, ring, acc_sh,
             *sems):
        gsems = sems[:NBUF]
        ssems = sems[NBUF:]
        cid = lax.axis_index("c")
        sid = lax.axis_index("s")
        gwid = cid * NS + sid
        base = sid * rpt
        bufs = [ring.at[pl.ds(bb * BE, BE)] for bb in range(NBUF)]

        zv = jnp.zeros((16,), jnp.float32)

        def fill_row(r, carry):
            for c in range(D // 16):
                ring[r, pl.ds(c * 16, 16)] = zv
            return carry

        lax.fori_loop(0, B, fill_row, 0)
        for t in range(chunks):
            pltpu.sync_copy(ring.at[pl.ds(0, B)],
                            acc_sh.at[pl.ds(base + t * B, B)])
        plsc.subcore_barrier()

        def outer(ci, carry):
            off = gwid * k64 + ci * CI2
            pltpu.sync_copy(src_hbm.at[pl.ds(off, CI2)], src_v)
            pltpu.sync_copy(dst_hbm.at[pl.ds(off, CI2)], dst_v)
            gath = [None] * NBUF
            scat = [None] * NBUF
            for s in range(2):
                gath[s] = pltpu.async_copy(
                    y_hbm.at[src_v.at[s]], bufs[s], gsems[s])
            for s in range(CI2):
                bb = s % NBUF
                gath[bb].wait()
                scat[bb] = pltpu.async_copy(
                    bufs[bb], acc_sh.at[dst_v.at[s]], ssems[bb], add=True)
                if s + 2 < CI2:
                    nb = (s + 2) % NBUF
                    if scat[nb] is not None:
                        scat[nb].wait()
                    gath[nb] = pltpu.async_copy(
                        y_hbm.at[src_v.at[s + 2]], bufs[nb], gsems[nb])
            for s in range(CI2 - NBUF, CI2):
                scat[s % NBUF].wait()
            return carry

        lax.fori_loop(0, n_ci, outer, 0)
        plsc.subcore_barrier()

        for t in range(chunks):
            pltpu.sync_copy(acc_sh.at[pl.ds(base + t * B, B)],
                            ring.at[pl.ds(0, B)])
            pltpu.sync_copy(ring.at[pl.ds(0, B)],
                            out_s.at[cid, pl.ds(base + t * B, B)])

    return pl.kernel(body, mesh=_sc_mesh(), out_type=out_types,
                     scratch_types=scratch_types)


def _make_sc_deg(acc_rows, k_steps):
    rpt = acc_rows // NS
    chunks = rpt // B
    n_ci = k_steps // CI

    out_types = [jax.ShapeDtypeStruct((NC, acc_rows, D), jnp.float32)]
    scratch_types = [
        pltpu.VMEM((CI, B), jnp.int32),
        pltpu.VMEM((B, D), jnp.float32),
        pltpu.VMEM_SHARED((acc_rows, D), jnp.float32),
    ]

    def body(dst_hbm, out_d, dst_v, buf_v, deg_sh):
        cid = lax.axis_index("c")
        sid = lax.axis_index("s")
        gwid = cid * NS + sid
        base = sid * rpt

        def fill(val):
            vv = jnp.full((16,), val, dtype=jnp.float32)

            def fill_row(r, carry):
                for c in range(D // 16):
                    buf_v[r, pl.ds(c * 16, 16)] = vv
                return carry

            lax.fori_loop(0, B, fill_row, 0)

        fill(0.0)
        for t in range(chunks):
            pltpu.sync_copy(buf_v, deg_sh.at[pl.ds(base + t * B, B)])
        fill(1.0)
        plsc.subcore_barrier()

        def outer(ci, carry):
            off = gwid * k_steps + ci * CI
            pltpu.sync_copy(dst_hbm.at[pl.ds(off, CI)], dst_v)
            for j in range(CI):
                pltpu.sync_copy(buf_v, deg_sh.at[dst_v.at[j]], add=True)
            return carry

        lax.fori_loop(0, n_ci, outer, 0)
        plsc.subcore_barrier()

        for t in range(chunks):
            pltpu.sync_copy(deg_sh.at[pl.ds(base + t * B, B)], buf_v)
            pltpu.sync_copy(buf_v, out_d.at[cid, pl.ds(base + t * B, B)])

    return pl.kernel(body, mesh=_sc_mesh(), out_type=out_types,
                     scratch_types=scratch_types)


def _pre_body(x_ref, wl_ref, wr_ref, b_ref, y_ref, z_ref):
    xb = x_ref[...]
    y_ref[...] = jnp.dot(xb, wl_ref[...], preferred_element_type=jnp.float32)
    z_ref[...] = (jnp.dot(xb, wr_ref[...], preferred_element_type=jnp.float32)
                  + b_ref[...])


def _mid_body(p0_ref, p1_ref, d0_ref, d1_ref, z_ref, wl_ref, wr_ref, b_ref,
              y_ref, zo_ref):
    deg = d0_ref[...][:, 0:1] + d1_ref[...][:, 0:1]
    inv = 1.0 / jnp.maximum(deg, 1.0)
    h = jnp.maximum((p0_ref[...] + p1_ref[...]) * inv + z_ref[...], 0.0)
    y_ref[...] = jnp.dot(h, wl_ref[...], preferred_element_type=jnp.float32)
    zo_ref[...] = (jnp.dot(h, wr_ref[...], preferred_element_type=jnp.float32)
                   + b_ref[...])


def _final_body(p0_ref, p1_ref, d0_ref, d1_ref, z_ref, o_ref):
    deg = d0_ref[...][:, 0:1] + d1_ref[...][:, 0:1]
    inv = 1.0 / jnp.maximum(deg, 1.0)
    o_ref[...] = (p0_ref[...] + p1_ref[...]) * inv + z_ref[...]


def _row_spec(r):
    return pl.BlockSpec((r, D), lambda i: (i, 0))


def _deg_spec(r):
    return pl.BlockSpec((r, D), lambda i: (i, 0))


def _w_spec():
    return pl.BlockSpec((D, D), lambda i: (0, 0))


def _b_spec():
    return pl.BlockSpec((1, D), lambda i: (0, 0))


def _tc_pre(x, wl, wr, b, blk):
    n = x.shape[0]
    return pl.pallas_call(
        _pre_body,
        grid=(n // blk,),
        in_specs=[_row_spec(blk), _w_spec(), _w_spec(), _b_spec()],
        out_specs=[_row_spec(blk), _row_spec(blk)],
        out_shape=[jax.ShapeDtypeStruct((n, D), jnp.float32)] * 2,
    )(x, wl, wr, b.reshape(1, D))


def _tc_mid(p0, p1, d0, d1, z, wl, wr, b, blk):
    n = z.shape[0]
    return pl.pallas_call(
        _mid_body,
        grid=(n // blk,),
        in_specs=[_row_spec(blk), _row_spec(blk), _deg_spec(blk),
                  _deg_spec(blk), _row_spec(blk), _w_spec(), _w_spec(),
                  _b_spec()],
        out_specs=[_row_spec(blk), _row_spec(blk)],
        out_shape=[jax.ShapeDtypeStruct((n, D), jnp.float32)] * 2,
    )(p0, p1, d0, d1, z, wl, wr, b.reshape(1, D))


def _tc_final(p0, p1, d0, d1, z, blk):
    n = z.shape[0]
    return pl.pallas_call(
        _final_body,
        grid=(n // blk,),
        in_specs=[_row_spec(blk), _row_spec(blk), _deg_spec(blk),
                  _deg_spec(blk), _row_spec(blk)],
        out_specs=_row_spec(blk),
        out_shape=jax.ShapeDtypeStruct((n, D), jnp.float32),
    )(p0, p1, d0, d1, z)


def kernel(x, edge_index, Wl1, Wr1, b1, Wl2, Wr2, b2, Wl3, Wr3, b3):
    n = x.shape[0]
    e = edge_index.shape[1]

    k_steps = -(-e // (NW * B))
    k_steps = -(-k_steps // CI) * CI
    e_pad = NW * B * k_steps
    k64 = e_pad // (NW * BE)
    acc_rows = -(-(n + 1) // (NS * B)) * (NS * B)
    blk = 1000 if n % 1000 == 0 else 8 * (n // 8)

    src = edge_index[0]
    dst = edge_index[1]
    pad = e_pad - e
    if pad:
        src = jnp.concatenate([src, jnp.zeros((pad,), jnp.int32)])
        dst = jnp.concatenate([dst, jnp.full((pad,), n, dtype=jnp.int32)])
    src64 = src.reshape(e_pad // BE, BE)
    dst64 = dst.reshape(e_pad // BE, BE)
    dst2 = dst.reshape(e_pad // B, B)

    sc_agg = _make_sc_agg(acc_rows, k64)
    sc_deg = _make_sc_deg(acc_rows, k_steps)

    (dp,) = sc_deg(dst2)
    d0 = dp[0, :n]
    d1 = dp[1, :n]

    y1, z1 = _tc_pre(x, Wl1, Wr1, b1, blk)
    (s1,) = sc_agg(y1, src64, dst64)

    y2, z2 = _tc_mid(s1[0, :n], s1[1, :n], d0, d1, z1, Wl2, Wr2, b2, blk)
    (s2,) = sc_agg(y2, src64, dst64)

    y3, z3 = _tc_mid(s2[0, :n], s2[1, :n], d0, d1, z2, Wl3, Wr3, b3, blk)
    (s3,) = sc_agg(y3, src64, dst64)

    return _tc_final(s3[0, :n], s3[1, :n], d0, d1, z3, blk)


# speedup vs baseline: 3.3994x; 1.0352x over previous
import jax
import jax.numpy as jnp
from jax import lax
from jax.experimental import pallas as pl
from jax.experimental.pallas import tpu as pltpu
from jax.experimental.pallas import tpu_sc as plsc

D = 128
B = 128
BE = 32
CI = 8
CI2 = 64
NBUF = 4
NC = 2
NS = 16
NW = NC * NS


def _sc_mesh():
    return plsc.VectorSubcoreMesh(core_axis_name="c", subcore_axis_name="s",
                                  num_cores=NC, num_subcores=NS)


def _chunks(rpt):
    out = []
    off = 0
    while off < rpt:
        out.append((off, min(B, rpt - off)))
        off += B
    return out


def _make_sc_agg(acc_rows, k64):
    rpt = acc_rows // NS
    n_ci = k64 // CI2

    out_types = [jax.ShapeDtypeStruct((NC, acc_rows, D), jnp.float32)]
    scratch_types = [
        pltpu.VMEM((CI2, BE), jnp.int32),
        pltpu.VMEM((CI2, BE), jnp.int32),
        pltpu.VMEM((NBUF * BE, D), jnp.float32),
        pltpu.VMEM_SHARED((acc_rows, D), jnp.float32),
    ] + [pltpu.SemaphoreType.DMA] * (2 * NBUF)

    def body(y_hbm, src_hbm, dst_hbm, out_s, src_v, dst_v, ring, acc_sh,
             *sems):
        gsems = sems[:NBUF]
        ssems = sems[NBUF:]
        cid = lax.axis_index("c")
        sid = lax.axis_index("s")
        gwid = cid * NS + sid
        base = sid * rpt
        bufs = [ring.at[pl.ds(bb * BE, BE)] for bb in range(NBUF)]

        zv = jnp.zeros((16,), jnp.float32)

        def fill_row(r, carry):
            for c in range(D // 16):
                ring[r, pl.ds(c * 16, 16)] = zv
            return carry

        lax.fori_loop(0, B, fill_row, 0)
        for off, sz in _chunks(rpt):
            pltpu.sync_copy(ring.at[pl.ds(0, sz)],
                            acc_sh.at[pl.ds(base + off, sz)])
        plsc.subcore_barrier()

        def outer(ci, carry):
            off = gwid * k64 + ci * CI2
            pltpu.sync_copy(src_hbm.at[pl.ds(off, CI2)], src_v)
            pltpu.sync_copy(dst_hbm.at[pl.ds(off, CI2)], dst_v)
            gath = [None] * NBUF
            scat = [None] * NBUF
            for s in range(2):
                gath[s] = pltpu.async_copy(
                    y_hbm.at[src_v.at[s]], bufs[s], gsems[s])
            for s in range(CI2):
                bb = s % NBUF
                gath[bb].wait()
                scat[bb] = pltpu.async_copy(
                    bufs[bb], acc_sh.at[dst_v.at[s]], ssems[bb], add=True)
                if s + 2 < CI2:
                    nb = (s + 2) % NBUF
                    if scat[nb] is not None:
                        scat[nb].wait()
                    gath[nb] = pltpu.async_copy(
                        y_hbm.at[src_v.at[s + 2]], bufs[nb], gsems[nb])
            for s in range(CI2 - NBUF, CI2):
                scat[s % NBUF].wait()
            return carry

        lax.fori_loop(0, n_ci, outer, 0)
        plsc.subcore_barrier()

        for off, sz in _chunks(rpt):
            pltpu.sync_copy(acc_sh.at[pl.ds(base + off, sz)],
                            ring.at[pl.ds(0, sz)])
            pltpu.sync_copy(ring.at[pl.ds(0, sz)],
                            out_s.at[cid, pl.ds(base + off, sz)])

    return pl.kernel(body, mesh=_sc_mesh(), out_type=out_types,
                     scratch_types=scratch_types)


def _make_sc_deg(acc_rows, k_steps):
    rpt = acc_rows // NS
    n_ci = k_steps // CI

    out_types = [jax.ShapeDtypeStruct((NC, acc_rows, D), jnp.float32)]
    scratch_types = [
        pltpu.VMEM((CI, B), jnp.int32),
        pltpu.VMEM((B, D), jnp.float32),
        pltpu.VMEM_SHARED((acc_rows, D), jnp.float32),
    ]

    def body(dst_hbm, out_d, dst_v, buf_v, deg_sh):
        cid = lax.axis_index("c")
        sid = lax.axis_index("s")
        gwid = cid * NS + sid
        base = sid * rpt

        def fill(val):
            vv = jnp.full((16,), val, dtype=jnp.float32)

            def fill_row(r, carry):
                for c in range(D // 16):
                    buf_v[r, pl.ds(c * 16, 16)] = vv
                return carry

            lax.fori_loop(0, B, fill_row, 0)

        fill(0.0)
        for off, sz in _chunks(rpt):
            pltpu.sync_copy(buf_v.at[pl.ds(0, sz)],
                            deg_sh.at[pl.ds(base + off, sz)])
        fill(1.0)
        plsc.subcore_barrier()

        def outer(ci, carry):
            off = gwid * k_steps + ci * CI
            pltpu.sync_copy(dst_hbm.at[pl.ds(off, CI)], dst_v)
            for j in range(CI):
                pltpu.sync_copy(buf_v, deg_sh.at[dst_v.at[j]], add=True)
            return carry

        lax.fori_loop(0, n_ci, outer, 0)
        plsc.subcore_barrier()

        for off, sz in _chunks(rpt):
            pltpu.sync_copy(deg_sh.at[pl.ds(base + off, sz)],
                            buf_v.at[pl.ds(0, sz)])
            pltpu.sync_copy(buf_v.at[pl.ds(0, sz)],
                            out_d.at[cid, pl.ds(base + off, sz)])

    return pl.kernel(body, mesh=_sc_mesh(), out_type=out_types,
                     scratch_types=scratch_types)


def _pre_body(x_ref, wl_ref, wr_ref, b_ref, y_ref, z_ref):
    xb = x_ref[...]
    y_ref[...] = jnp.dot(xb, wl_ref[...], preferred_element_type=jnp.float32)
    z_ref[...] = (jnp.dot(xb, wr_ref[...], preferred_element_type=jnp.float32)
                  + b_ref[...])


def _mid_body(p0_ref, p1_ref, d0_ref, d1_ref, z_ref, wl_ref, wr_ref, b_ref,
              y_ref, zo_ref):
    deg = d0_ref[...][:, 0:1] + d1_ref[...][:, 0:1]
    inv = 1.0 / jnp.maximum(deg, 1.0)
    h = jnp.maximum((p0_ref[...] + p1_ref[...]) * inv + z_ref[...], 0.0)
    y_ref[...] = jnp.dot(h, wl_ref[...], preferred_element_type=jnp.float32)
    zo_ref[...] = (jnp.dot(h, wr_ref[...], preferred_element_type=jnp.float32)
                   + b_ref[...])


def _final_body(p0_ref, p1_ref, d0_ref, d1_ref, z_ref, o_ref):
    deg = d0_ref[...][:, 0:1] + d1_ref[...][:, 0:1]
    inv = 1.0 / jnp.maximum(deg, 1.0)
    o_ref[...] = (p0_ref[...] + p1_ref[...]) * inv + z_ref[...]


def _row_spec(r):
    return pl.BlockSpec((r, D), lambda i: (i, 0))


def _deg_spec(r):
    return pl.BlockSpec((r, D), lambda i: (i, 0))


def _w_spec():
    return pl.BlockSpec((D, D), lambda i: (0, 0))


def _b_spec():
    return pl.BlockSpec((1, D), lambda i: (0, 0))


def _tc_pre(x, wl, wr, b, blk):
    n = x.shape[0]
    return pl.pallas_call(
        _pre_body,
        grid=(n // blk,),
        in_specs=[_row_spec(blk), _w_spec(), _w_spec(), _b_spec()],
        out_specs=[_row_spec(blk), _row_spec(blk)],
        out_shape=[jax.ShapeDtypeStruct((n, D), jnp.float32)] * 2,
    )(x, wl, wr, b.reshape(1, D))


def _tc_mid(p0, p1, d0, d1, z, wl, wr, b, blk):
    n = z.shape[0]
    return pl.pallas_call(
        _mid_body,
        grid=(n // blk,),
        in_specs=[_row_spec(blk), _row_spec(blk), _deg_spec(blk),
                  _deg_spec(blk), _row_spec(blk), _w_spec(), _w_spec(),
                  _b_spec()],
        out_specs=[_row_spec(blk), _row_spec(blk)],
        out_shape=[jax.ShapeDtypeStruct((n, D), jnp.float32)] * 2,
    )(p0, p1, d0, d1, z, wl, wr, b.reshape(1, D))


def _tc_final(p0, p1, d0, d1, z, blk):
    n = z.shape[0]
    return pl.pallas_call(
        _final_body,
        grid=(n // blk,),
        in_specs=[_row_spec(blk), _row_spec(blk), _deg_spec(blk),
                  _deg_spec(blk), _row_spec(blk)],
        out_specs=_row_spec(blk),
        out_shape=jax.ShapeDtypeStruct((n, D), jnp.float32),
    )(p0, p1, d0, d1, z)


def kernel(x, edge_index, Wl1, Wr1, b1, Wl2, Wr2, b2, Wl3, Wr3, b3):
    n = x.shape[0]
    e = edge_index.shape[1]

    k_steps = -(-e // (NW * B))
    k_steps = -(-k_steps // CI) * CI
    e_pad = NW * B * k_steps
    k64 = e_pad // (NW * BE)
    acc_rows = -(-(n + 1) // (NS * 8)) * (NS * 8)
    blk = 1000 if n % 1000 == 0 else 8 * (n // 8)

    src = edge_index[0]
    dst = edge_index[1]
    pad = e_pad - e
    if pad:
        src = jnp.concatenate([src, jnp.zeros((pad,), jnp.int32)])
        dst = jnp.concatenate([dst, jnp.full((pad,), n, dtype=jnp.int32)])
    src64 = src.reshape(e_pad // BE, BE)
    dst64 = dst.reshape(e_pad // BE, BE)
    dst2 = dst.reshape(e_pad // B, B)

    sc_agg = _make_sc_agg(acc_rows, k64)
    sc_deg = _make_sc_deg(acc_rows, k_steps)

    (dp,) = sc_deg(dst2)
    d0 = dp[0, :n]
    d1 = dp[1, :n]

    y1, z1 = _tc_pre(x, Wl1, Wr1, b1, blk)
    (s1,) = sc_agg(y1, src64, dst64)

    y2, z2 = _tc_mid(s1[0, :n], s1[1, :n], d0, d1, z1, Wl2, Wr2, b2, blk)
    (s2,) = sc_agg(y2, src64, dst64)

    y3, z3 = _tc_mid(s2[0, :n], s2[1, :n], d0, d1, z2, Wl3, Wr3, b3, blk)
    (s3,) = sc_agg(y3, src64, dst64)

    return _tc_final(s3[0, :n], s3[1, :n], d0, d1, z3, blk)
